# Initial kernel scaffold; baseline (speedup 1.0000x reference)
import jax
import jax.numpy as jnp
from jax.experimental import pallas as pl


def kernel(x, edge_index, edge_attr, u, batch, W1, b1, W2, b2, W3, b3):
    raise NotImplementedError("write your pallas kernel here")


# speedup vs baseline: 58.8647x; 58.8647x over previous
import functools

import jax
import jax.numpy as jnp
from jax import lax
from jax.experimental import pallas as pl
from jax.experimental.pallas import tpu as pltpu
from jax.experimental.pallas import tpu_sc as plsc

_NC = 2
_NS = 16
_NW = _NC * _NS

_GRP = 128
_GPC = 16
_CHUNK = _GRP * _GPC
_DW = 8


def _sc_segment_sum(row2d, col2d, x_aug, zeros_stripe, n_pad):
    g_total = row2d.shape[0]
    n_chunks = g_total // _GPC
    rows_per_tile = n_pad // _NS

    mesh = plsc.VectorSubcoreMesh(core_axis_name="c", subcore_axis_name="s")

    @functools.partial(
        pl.kernel,
        out_type=jax.ShapeDtypeStruct((_NC, n_pad, _DW), jnp.float32),
        mesh=mesh,
        compiler_params=pltpu.CompilerParams(use_tc_tiling_on_sc=False),
        scratch_types=[
            pltpu.VMEM((_GPC, _GRP), jnp.int32),
            pltpu.VMEM((_GPC, _GRP), jnp.int32),
            pltpu.VMEM((_CHUNK, _DW), jnp.float32),
            pltpu.VMEM_SHARED((n_pad, _DW), jnp.float32),
            pltpu.SemaphoreType.DMA,
            pltpu.SemaphoreType.DMA,
            pltpu.SemaphoreType.DMA,
        ],
    )
    def kern(row_hbm, col_hbm, xaug_hbm, zero_hbm, out_hbm,
             row_v, col_v, rows_v, acc_sh, sem_i, sem_g, sem_s):
        cid = lax.axis_index("c")
        sid = lax.axis_index("s")
        wid = sid * _NC + cid

        pltpu.sync_copy(zero_hbm,
                        acc_sh.at[pl.ds(sid * rows_per_tile, rows_per_tile), :])
        plsc.subcore_barrier()

        n_my = (n_chunks - wid + _NW - 1) // _NW

        def body(i, carry):
            c = wid + i * _NW
            base = c * _GPC
            cp_r = pltpu.async_copy(row_hbm.at[pl.ds(base, _GPC), :], row_v,
                                    sem_i)
            cp_c = pltpu.async_copy(col_hbm.at[pl.ds(base, _GPC), :], col_v,
                                    sem_i)
            cp_r.wait()
            cp_c.wait()
            gathers = [
                pltpu.async_copy(xaug_hbm.at[row_v.at[k]],
                                 rows_v.at[pl.ds(k * _GRP, _GRP), :], sem_g)
                for k in range(_GPC)
            ]
            for g in gathers:
                g.wait()
            scatters = [
                pltpu.async_copy(rows_v.at[pl.ds(k * _GRP, _GRP), :],
                                 acc_sh.at[col_v.at[k]], sem_s, add=True)
                for k in range(_GPC)
            ]
            for s in scatters:
                s.wait()
            return carry

        lax.fori_loop(0, n_my, body, 0)

        plsc.subcore_barrier()
        pltpu.sync_copy(acc_sh.at[pl.ds(sid * rows_per_tile, rows_per_tile), :],
                        out_hbm.at[cid,
                                   pl.ds(sid * rows_per_tile, rows_per_tile),
                                   :])

    return kern(row2d, col2d, x_aug, zeros_stripe)


def _mlp_body(x_ref, acc_ref, w1_ref, b1_ref, w2_ref, b2_ref, w3_ref, b3_ref,
              o_ref):
    s = acc_ref[0] + acc_ref[1]
    cnt = jnp.maximum(s[:, 5:6], 1.0)
    mean = s[:, :5] / cnt
    w1 = w1_ref[...]
    h = x_ref[...] @ w1[:5] + mean @ w1[5:] + b1_ref[...]
    h = jnp.maximum(h, 0.0)
    h = jnp.maximum(h @ w2_ref[...] + b2_ref[...], 0.0)
    o_ref[...] = h @ w3_ref[...] + b3_ref[...]


def kernel(x, edge_index, edge_attr, u, batch, W1, b1, W2, b2, W3, b3):
    n = x.shape[0]
    e = edge_index.shape[1]
    f = x.shape[1]

    n_pad = ((n + _NS * 8 - 1) // (_NS * 8)) * (_NS * 8)

    row2d = edge_index[0].astype(jnp.int32).reshape(e // _GRP, _GRP)
    col2d = edge_index[1].astype(jnp.int32).reshape(e // _GRP, _GRP)

    x_aug = jnp.concatenate(
        [x, jnp.ones((n, 1), jnp.float32), jnp.zeros((n, _DW - f - 1),
                                                     jnp.float32)], axis=1)
    x_aug = jnp.pad(x_aug, ((0, n_pad - n), (0, 0)))
    zeros_stripe = jnp.zeros((n_pad // _NS, _DW), jnp.float32)

    acc = _sc_segment_sum(row2d, col2d, x_aug, zeros_stripe, n_pad)

    blk = 800
    grid = n // blk
    out = pl.pallas_call(
        _mlp_body,
        grid=(grid,),
        in_specs=[
            pl.BlockSpec((blk, f), lambda i: (i, 0)),
            pl.BlockSpec((_NC, blk, _DW), lambda i: (0, i, 0)),
            pl.BlockSpec(W1.shape, lambda i: (0, 0)),
            pl.BlockSpec(b1.shape, lambda i: (0,)),
            pl.BlockSpec(W2.shape, lambda i: (0, 0)),
            pl.BlockSpec(b2.shape, lambda i: (0,)),
            pl.BlockSpec(W3.shape, lambda i: (0, 0)),
            pl.BlockSpec(b3.shape, lambda i: (0,)),
        ],
        out_specs=pl.BlockSpec((blk, W3.shape[1]), lambda i: (i, 0)),
        out_shape=jax.ShapeDtypeStruct((n, W3.shape[1]), jnp.float32),
    )(x, acc, W1, b1, W2, b2, W3, b3)
    return out


# speedup vs baseline: 71.9089x; 1.2216x over previous
import functools

import jax
import jax.numpy as jnp
from jax import lax
from jax.experimental import pallas as pl
from jax.experimental.pallas import tpu as pltpu
from jax.experimental.pallas import tpu_sc as plsc

_NC = 2
_NS = 16
_NW = _NC * _NS

_GRP = 128
_GPC = 16
_CHUNK = _GRP * _GPC
_DW = 8


def _sc_segment_sum(ei3, x_aug, zeros_stripe, n_pad):
    g_total = ei3.shape[1]
    n_chunks = g_total // _GPC
    rows_per_tile = n_pad // _NS

    mesh = plsc.VectorSubcoreMesh(core_axis_name="c", subcore_axis_name="s")

    @functools.partial(
        pl.kernel,
        out_type=jax.ShapeDtypeStruct((_NC, n_pad, _DW), jnp.float32),
        mesh=mesh,
        compiler_params=pltpu.CompilerParams(use_tc_tiling_on_sc=False),
        scratch_types=[
            pltpu.VMEM((_GPC, _GRP), jnp.int32),
            pltpu.VMEM((_GPC, _GRP), jnp.int32),
            pltpu.VMEM((_CHUNK, _DW), jnp.float32),
            pltpu.VMEM((_GPC, _GRP), jnp.int32),
            pltpu.VMEM((_GPC, _GRP), jnp.int32),
            pltpu.VMEM((_CHUNK, _DW), jnp.float32),
            pltpu.VMEM_SHARED((n_pad, _DW), jnp.float32),
            pltpu.SemaphoreType.DMA,
            pltpu.SemaphoreType.DMA,
            pltpu.SemaphoreType.DMA,
            pltpu.SemaphoreType.DMA,
            pltpu.SemaphoreType.DMA,
            pltpu.SemaphoreType.DMA,
        ],
    )
    def kern(ei_hbm, xaug_hbm, zero_hbm, out_hbm,
             row_a, col_a, rows_a, row_b, col_b, rows_b, acc_sh,
             sem_ia, sem_ib, sem_ga, sem_gb, sem_sa, sem_sb):
        cid = lax.axis_index("c")
        sid = lax.axis_index("s")
        wid = sid * _NC + cid

        pltpu.sync_copy(zero_hbm,
                        acc_sh.at[pl.ds(sid * rows_per_tile, rows_per_tile), :])
        plsc.subcore_barrier()

        n_my = (n_chunks - wid + _NW - 1) // _NW

        def load_idx(c, row_v, col_v, sem):
            base = c * _GPC
            return (pltpu.async_copy(ei_hbm.at[0, pl.ds(base, _GPC), :],
                                     row_v, sem),
                    pltpu.async_copy(ei_hbm.at[1, pl.ds(base, _GPC), :],
                                     col_v, sem))

        def gathers(row_v, rows_v, sem):
            return [
                pltpu.async_copy(xaug_hbm.at[row_v.at[k]],
                                 rows_v.at[pl.ds(k * _GRP, _GRP), :], sem)
                for k in range(_GPC)
            ]

        def scatters(rows_v, col_v, sem):
            return [
                pltpu.async_copy(rows_v.at[pl.ds(k * _GRP, _GRP), :],
                                 acc_sh.at[col_v.at[k]], sem, add=True)
                for k in range(_GPC)
            ]

        def body(i, carry):
            ca = wid + (2 * i) * _NW
            cb = wid + (2 * i + 1) * _NW
            ira, ica = load_idx(ca, row_a, col_a, sem_ia)
            irb, icb = load_idx(cb, row_b, col_b, sem_ib)
            ira.wait()
            ica.wait()
            ga = gathers(row_a, rows_a, sem_ga)
            for g in ga:
                g.wait()
            sa = scatters(rows_a, col_a, sem_sa)
            irb.wait()
            icb.wait()
            gb = gathers(row_b, rows_b, sem_gb)
            for s in sa:
                s.wait()
            for g in gb:
                g.wait()
            sb = scatters(rows_b, col_b, sem_sb)
            for s in sb:
                s.wait()
            return carry

        lax.fori_loop(0, n_my // 2, body, 0)

        @pl.when(n_my % 2 == 1)
        def _tail():
            c = wid + (n_my - 1) * _NW
            ir, ic = load_idx(c, row_a, col_a, sem_ia)
            ir.wait()
            ic.wait()
            ga = gathers(row_a, rows_a, sem_ga)
            for g in ga:
                g.wait()
            sa = scatters(rows_a, col_a, sem_sa)
            for s in sa:
                s.wait()

        plsc.subcore_barrier()
        pltpu.sync_copy(acc_sh.at[pl.ds(sid * rows_per_tile, rows_per_tile), :],
                        out_hbm.at[cid,
                                   pl.ds(sid * rows_per_tile, rows_per_tile),
                                   :])

    return kern(ei3, x_aug, zeros_stripe)


def _mlp_body(x_ref, acc_ref, w1_ref, b1_ref, w2_ref, b2_ref, w3_ref, b3_ref,
              o_ref):
    s = acc_ref[0] + acc_ref[1]
    cnt = jnp.maximum(s[:, 5:6], 1.0)
    mean = s[:, :5] / cnt
    w1 = w1_ref[...]
    h = x_ref[...] @ w1[:5] + mean @ w1[5:] + b1_ref[...]
    h = jnp.maximum(h, 0.0)
    h = jnp.maximum(h @ w2_ref[...] + b2_ref[...], 0.0)
    o_ref[...] = h @ w3_ref[...] + b3_ref[...]


def kernel(x, edge_index, edge_attr, u, batch, W1, b1, W2, b2, W3, b3):
    n = x.shape[0]
    e = edge_index.shape[1]
    f = x.shape[1]

    n_pad = ((n + _NS * 8 - 1) // (_NS * 8)) * (_NS * 8)

    ei3 = edge_index.astype(jnp.int32).reshape(2, e // _GRP, _GRP)

    x_aug = jnp.concatenate(
        [x, jnp.ones((n, 1), jnp.float32), jnp.zeros((n, _DW - f - 1),
                                                     jnp.float32)], axis=1)
    x_aug = jnp.pad(x_aug, ((0, n_pad - n), (0, 0)))
    zeros_stripe = jnp.zeros((n_pad // _NS, _DW), jnp.float32)

    acc = _sc_segment_sum(ei3, x_aug, zeros_stripe, n_pad)

    blk = 4000
    grid = n // blk
    out = pl.pallas_call(
        _mlp_body,
        grid=(grid,),
        in_specs=[
            pl.BlockSpec((blk, f), lambda i: (i, 0)),
            pl.BlockSpec((_NC, blk, _DW), lambda i: (0, i, 0)),
            pl.BlockSpec(W1.shape, lambda i: (0, 0)),
            pl.BlockSpec(b1.shape, lambda i: (0,)),
            pl.BlockSpec(W2.shape, lambda i: (0, 0)),
            pl.BlockSpec(b2.shape, lambda i: (0,)),
            pl.BlockSpec(W3.shape, lambda i: (0, 0)),
            pl.BlockSpec(b3.shape, lambda i: (0,)),
        ],
        out_specs=pl.BlockSpec((blk, W3.shape[1]), lambda i: (i, 0)),
        out_shape=jax.ShapeDtypeStruct((n, W3.shape[1]), jnp.float32),
    )(x, acc, W1, b1, W2, b2, W3, b3)
    return out


# speedup vs baseline: 90.3953x; 1.2571x over previous
import functools

import jax
import jax.numpy as jnp
from jax import lax
from jax.experimental import pallas as pl
from jax.experimental.pallas import tpu as pltpu
from jax.experimental.pallas import tpu_sc as plsc

_NC = 2
_NS = 16
_NW = _NC * _NS

_GRP = 128
_GPC = 8
_CHUNK = _GRP * _GPC
_DW = 8


def _sc_segment_sum(ei3, x_aug, zeros_stripe, n_pad):
    g_total = ei3.shape[1]
    n_chunks = g_total // _GPC
    rows_per_tile = n_pad // _NS

    mesh = plsc.VectorSubcoreMesh(core_axis_name="c", subcore_axis_name="s")

    @functools.partial(
        pl.kernel,
        out_type=jax.ShapeDtypeStruct((_NC, n_pad, _DW), jnp.float32),
        mesh=mesh,
        compiler_params=pltpu.CompilerParams(use_tc_tiling_on_sc=False),
        scratch_types=[
            pltpu.VMEM((_GPC, _GRP), jnp.int32),
            pltpu.VMEM((_GPC, _GRP), jnp.int32),
            pltpu.VMEM((_CHUNK, _DW), jnp.float32),
            pltpu.VMEM((_GPC, _GRP), jnp.int32),
            pltpu.VMEM((_GPC, _GRP), jnp.int32),
            pltpu.VMEM((_CHUNK, _DW), jnp.float32),
            pltpu.VMEM_SHARED((n_pad, _DW), jnp.float32),
            pltpu.VMEM_SHARED((n_pad, _DW), jnp.float32),
            pltpu.SemaphoreType.DMA,
            pltpu.SemaphoreType.DMA,
            pltpu.SemaphoreType.DMA,
            pltpu.SemaphoreType.DMA,
            pltpu.SemaphoreType.DMA,
            pltpu.SemaphoreType.DMA,
        ],
    )
    def kern(ei_hbm, xaug_hbm, zero_hbm, out_hbm,
             row_a, col_a, rows_a, row_b, col_b, rows_b, acc_sh, tab_sh,
             sem_ia, sem_ib, sem_ga, sem_gb, sem_sa, sem_sb):
        cid = lax.axis_index("c")
        sid = lax.axis_index("s")
        wid = sid * _NC + cid

        stripe = pl.ds(sid * rows_per_tile, rows_per_tile)
        pltpu.sync_copy(zero_hbm, acc_sh.at[stripe, :])
        pltpu.sync_copy(xaug_hbm.at[stripe, :], tab_sh.at[stripe, :])
        plsc.subcore_barrier()

        n_my = (n_chunks - wid + _NW - 1) // _NW

        def load_idx(c, row_v, col_v, sem):
            base = c * _GPC
            return (pltpu.async_copy(ei_hbm.at[0, pl.ds(base, _GPC), :],
                                     row_v, sem),
                    pltpu.async_copy(ei_hbm.at[1, pl.ds(base, _GPC), :],
                                     col_v, sem))

        def gathers(row_v, rows_v, sem):
            return [
                pltpu.async_copy(tab_sh.at[row_v.at[k]],
                                 rows_v.at[pl.ds(k * _GRP, _GRP), :], sem)
                for k in range(_GPC)
            ]

        def scatters(rows_v, col_v, sem):
            return [
                pltpu.async_copy(rows_v.at[pl.ds(k * _GRP, _GRP), :],
                                 acc_sh.at[col_v.at[k]], sem, add=True)
                for k in range(_GPC)
            ]

        def body(i, carry):
            ca = wid + (2 * i) * _NW
            cb = wid + (2 * i + 1) * _NW
            ira, ica = load_idx(ca, row_a, col_a, sem_ia)
            irb, icb = load_idx(cb, row_b, col_b, sem_ib)
            ira.wait()
            ica.wait()
            ga = gathers(row_a, rows_a, sem_ga)
            for g in ga:
                g.wait()
            sa = scatters(rows_a, col_a, sem_sa)
            irb.wait()
            icb.wait()
            gb = gathers(row_b, rows_b, sem_gb)
            for s in sa:
                s.wait()
            for g in gb:
                g.wait()
            sb = scatters(rows_b, col_b, sem_sb)
            for s in sb:
                s.wait()
            return carry

        lax.fori_loop(0, n_my // 2, body, 0)

        @pl.when(n_my % 2 == 1)
        def _tail():
            c = wid + (n_my - 1) * _NW
            ir, ic = load_idx(c, row_a, col_a, sem_ia)
            ir.wait()
            ic.wait()
            ga = gathers(row_a, rows_a, sem_ga)
            for g in ga:
                g.wait()
            sa = scatters(rows_a, col_a, sem_sa)
            for s in sa:
                s.wait()

        plsc.subcore_barrier()
        pltpu.sync_copy(acc_sh.at[pl.ds(sid * rows_per_tile, rows_per_tile), :],
                        out_hbm.at[cid,
                                   pl.ds(sid * rows_per_tile, rows_per_tile),
                                   :])

    return kern(ei3, x_aug, zeros_stripe)


def _mlp_body(x_ref, acc_ref, w1_ref, b1_ref, w2_ref, b2_ref, w3_ref, b3_ref,
              o_ref):
    s = acc_ref[0] + acc_ref[1]
    cnt = jnp.maximum(s[:, 5:6], 1.0)
    mean = s[:, :5] / cnt
    w1 = w1_ref[...]
    h = x_ref[...] @ w1[:5] + mean @ w1[5:] + b1_ref[...]
    h = jnp.maximum(h, 0.0)
    h = jnp.maximum(h @ w2_ref[...] + b2_ref[...], 0.0)
    o_ref[...] = h @ w3_ref[...] + b3_ref[...]


def kernel(x, edge_index, edge_attr, u, batch, W1, b1, W2, b2, W3, b3):
    n = x.shape[0]
    e = edge_index.shape[1]
    f = x.shape[1]

    n_pad = ((n + _NS * 8 - 1) // (_NS * 8)) * (_NS * 8)

    ei3 = edge_index.astype(jnp.int32).reshape(2, e // _GRP, _GRP)

    x_aug = jnp.concatenate(
        [x, jnp.ones((n, 1), jnp.float32), jnp.zeros((n, _DW - f - 1),
                                                     jnp.float32)], axis=1)
    x_aug = jnp.pad(x_aug, ((0, n_pad - n), (0, 0)))
    zeros_stripe = jnp.zeros((n_pad // _NS, _DW), jnp.float32)

    acc = _sc_segment_sum(ei3, x_aug, zeros_stripe, n_pad)

    blk = 4000
    grid = n // blk
    out = pl.pallas_call(
        _mlp_body,
        grid=(grid,),
        in_specs=[
            pl.BlockSpec((blk, f), lambda i: (i, 0)),
            pl.BlockSpec((_NC, blk, _DW), lambda i: (0, i, 0)),
            pl.BlockSpec(W1.shape, lambda i: (0, 0)),
            pl.BlockSpec(b1.shape, lambda i: (0,)),
            pl.BlockSpec(W2.shape, lambda i: (0, 0)),
            pl.BlockSpec(b2.shape, lambda i: (0,)),
            pl.BlockSpec(W3.shape, lambda i: (0, 0)),
            pl.BlockSpec(b3.shape, lambda i: (0,)),
        ],
        out_specs=pl.BlockSpec((blk, W3.shape[1]), lambda i: (i, 0)),
        out_shape=jax.ShapeDtypeStruct((n, W3.shape[1]), jnp.float32),
    )(x, acc, W1, b1, W2, b2, W3, b3)
    return out


# speedup vs baseline: 149.2569x; 1.6512x over previous
import functools

import jax
import jax.numpy as jnp
from jax import lax
from jax.experimental import pallas as pl
from jax.experimental.pallas import tpu as pltpu
from jax.experimental.pallas import tpu_sc as plsc

_NC = 2
_NS = 16
_NW = _NC * _NS

_GRP = 128
_GPC = 8
_CHUNK = _GRP * _GPC
_DW = 8


_PIECES = 17
_PGROUPS = 23


def _sc_segment_sum(ei_t, x_fm, zeros_stripe, n_pad, f):
    g_total = ei_t.shape[0]
    n_chunks = g_total // _GPC
    rows_per_tile = n_pad // _NS
    piece = rows_per_tile // _PIECES

    mesh = plsc.VectorSubcoreMesh(core_axis_name="c", subcore_axis_name="s")

    @functools.partial(
        pl.kernel,
        out_type=(jax.ShapeDtypeStruct((_NC, n_pad, _DW), jnp.float32),
                  jax.ShapeDtypeStruct((n_pad, _DW), jnp.float32)),
        mesh=mesh,
        compiler_params=pltpu.CompilerParams(use_tc_tiling_on_sc=False,
                                             needs_layout_passes=False),
        scratch_types=[
            pltpu.VMEM((_GPC, _GRP), jnp.int32),
            pltpu.VMEM((_GPC, _GRP), jnp.int32),
            pltpu.VMEM((_CHUNK, _DW), jnp.float32),
            pltpu.VMEM((_GPC, _GRP), jnp.int32),
            pltpu.VMEM((_GPC, _GRP), jnp.int32),
            pltpu.VMEM((_CHUNK, _DW), jnp.float32),
            pltpu.VMEM((f, piece), jnp.float32),
            pltpu.VMEM((piece, _DW), jnp.float32),
            pltpu.VMEM_SHARED((n_pad, _DW), jnp.float32),
            pltpu.VMEM_SHARED((n_pad, _DW), jnp.float32),
            pltpu.SemaphoreType.DMA,
            pltpu.SemaphoreType.DMA,
            pltpu.SemaphoreType.DMA,
            pltpu.SemaphoreType.DMA,
            pltpu.SemaphoreType.DMA,
            pltpu.SemaphoreType.DMA,
        ],
    )
    def kern(ei_hbm, x_hbm, zero_hbm, acc_hbm, tab_hbm,
             row_a, col_a, rows_a, row_b, col_b, rows_b, xv, tabb,
             acc_sh, tab_sh,
             sem_ia, sem_ib, sem_ga, sem_gb, sem_sa, sem_sb):
        cid = lax.axis_index("c")
        sid = lax.axis_index("s")
        wid = sid * _NC + cid

        stripe = pl.ds(sid * rows_per_tile, rows_per_tile)
        pltpu.sync_copy(zero_hbm, acc_sh.at[stripe, :])
        iota = lax.iota(jnp.int32, 16)
        ones_v = jnp.ones((16,), jnp.float32)
        zeros_v = jnp.zeros((16,), jnp.float32)
        for p in range(_PIECES):
            node0 = sid * rows_per_tile + p * piece
            pltpu.sync_copy(x_hbm.at[:, pl.ds(node0, piece)], xv)

            def bgroup(g, carry):
                ln = g * 16 + iota
                for j in range(f):
                    v = xv[j, pl.ds(g * 16, 16)]
                    plsc.store_scatter(tabb, [ln, iota * 0 + j], v)
                plsc.store_scatter(tabb, [ln, iota * 0 + f], ones_v)
                for j in range(f + 1, _DW):
                    plsc.store_scatter(tabb, [ln, iota * 0 + j], zeros_v)
                return carry

            lax.fori_loop(0, _PGROUPS, bgroup, 0)
            pltpu.sync_copy(tabb, tab_sh.at[pl.ds(node0, piece), :])
        plsc.subcore_barrier()

        n_my = (n_chunks - wid + _NW - 1) // _NW

        def load_idx(c, row_v, col_v, sem):
            base = c * _GPC
            return (pltpu.async_copy(ei_hbm.at[pl.ds(base, _GPC), 0, :],
                                     row_v, sem),
                    pltpu.async_copy(ei_hbm.at[pl.ds(base, _GPC), 1, :],
                                     col_v, sem))

        def gathers(row_v, rows_v, sem):
            return [
                pltpu.async_copy(tab_sh.at[row_v.at[k]],
                                 rows_v.at[pl.ds(k * _GRP, _GRP), :], sem)
                for k in range(_GPC)
            ]

        def scatters(rows_v, col_v, sem):
            return [
                pltpu.async_copy(rows_v.at[pl.ds(k * _GRP, _GRP), :],
                                 acc_sh.at[col_v.at[k]], sem, add=True)
                for k in range(_GPC)
            ]

        def body(i, carry):
            ca = wid + (2 * i) * _NW
            cb = wid + (2 * i + 1) * _NW
            ira, ica = load_idx(ca, row_a, col_a, sem_ia)
            irb, icb = load_idx(cb, row_b, col_b, sem_ib)
            ira.wait()
            ica.wait()
            ga = gathers(row_a, rows_a, sem_ga)
            for g in ga:
                g.wait()
            sa = scatters(rows_a, col_a, sem_sa)
            irb.wait()
            icb.wait()
            gb = gathers(row_b, rows_b, sem_gb)
            for s in sa:
                s.wait()
            for g in gb:
                g.wait()
            sb = scatters(rows_b, col_b, sem_sb)
            for s in sb:
                s.wait()
            return carry

        lax.fori_loop(0, n_my // 2, body, 0)

        @pl.when(n_my % 2 == 1)
        def _tail():
            c = wid + (n_my - 1) * _NW
            ir, ic = load_idx(c, row_a, col_a, sem_ia)
            ir.wait()
            ic.wait()
            ga = gathers(row_a, rows_a, sem_ga)
            for g in ga:
                g.wait()
            sa = scatters(rows_a, col_a, sem_sa)
            for s in sa:
                s.wait()

        plsc.subcore_barrier()
        pltpu.sync_copy(acc_sh.at[stripe, :], acc_hbm.at[cid, stripe, :])

        @pl.when(cid == 0)
        def _dump_tab():
            pltpu.sync_copy(tab_sh.at[stripe, :], tab_hbm.at[stripe, :])

    return kern(ei_t, x_fm, zeros_stripe)


def _packed_weights(W1, b1, W2, b2, W3, b3):
    f = W1.shape[0] - W3.shape[1]
    eye = jnp.eye(16, dtype=jnp.float32)
    z8 = jnp.zeros((_DW, _DW), jnp.float32)
    w1x = jnp.kron(eye, z8.at[:f, :].set(W1[:f]))
    w1m = jnp.kron(eye, z8.at[:f, :].set(W1[f:]))
    sel = jnp.kron(eye, z8.at[f, :].set(1.0))
    w2d = jnp.kron(eye, W2)
    w3d = jnp.kron(eye, z8.at[:, :W3.shape[1]].set(W3))
    b1p = jnp.tile(b1, 16)
    b2p = jnp.tile(b2, 16)
    b3p = jnp.tile(jnp.pad(b3, (0, _DW - W3.shape[1])), 16)
    return w1x, w1m, sel, w2d, w3d, b1p, b2p, b3p


def _mlp_body(acc_ref, tab_ref, w1x_ref, w1m_ref, sel_ref, w2_ref, w3_ref,
              b1_ref, b2_ref, b3_ref, o_ref):
    s = acc_ref[0] + acc_ref[1]
    cnt = jnp.maximum(s @ sel_ref[...], 1.0)
    mean = s / cnt
    h = tab_ref[...] @ w1x_ref[...] + mean @ w1m_ref[...] + b1_ref[...]
    h = jnp.maximum(h, 0.0)
    h = jnp.maximum(h @ w2_ref[...] + b2_ref[...], 0.0)
    o_ref[...] = h @ w3_ref[...] + b3_ref[...]


def kernel(x, edge_index, edge_attr, u, batch, W1, b1, W2, b2, W3, b3):
    n = x.shape[0]
    e = edge_index.shape[1]
    f = x.shape[1]

    n_pad = ((n + _NS * 8 - 1) // (_NS * 8)) * (_NS * 8)
    g_total = e // _GRP
    rpt = n_pad // _NS

    ei_t = jnp.transpose(
        edge_index.astype(jnp.int32).reshape(2, g_total, _GRP), (1, 0, 2))

    x_fm = jnp.pad(x.T, ((0, 0), (0, n_pad - n)))
    zeros_stripe = jnp.zeros((rpt, _DW), jnp.float32)

    acc, tab = _sc_segment_sum(ei_t, x_fm, zeros_stripe, n_pad, f)

    pk_rows = n_pad * _DW // 128
    acc_pk = acc.reshape(_NC, pk_rows, 128)
    tab_pk = tab.reshape(pk_rows, 128)

    pw = _packed_weights(W1, b1, W2, b2, W3, b3)

    blk = 368
    grid = pk_rows // blk
    wspec = pl.BlockSpec((128, 128), lambda i: (0, 0))
    bspec = pl.BlockSpec((128,), lambda i: (0,))
    out_pk = pl.pallas_call(
        _mlp_body,
        grid=(grid,),
        in_specs=[
            pl.BlockSpec((_NC, blk, 128), lambda i: (0, i, 0)),
            pl.BlockSpec((blk, 128), lambda i: (i, 0)),
            wspec, wspec, wspec, wspec, wspec,
            bspec, bspec, bspec,
        ],
        out_specs=pl.BlockSpec((blk, 128), lambda i: (i, 0)),
        out_shape=jax.ShapeDtypeStruct((pk_rows, 128), jnp.float32),
    )(acc_pk, tab_pk, *pw)

    out_fm = jnp.transpose(out_pk.reshape(pk_rows, 16, _DW),
                           (2, 0, 1)).reshape(_DW, n_pad)
    return out_fm[:W3.shape[1], :n].T
